# baseline (speedup 1.0000x reference)
import functools

import jax
import jax.numpy as jnp
import numpy as np
from jax import lax
from jax.experimental import pallas as pl
from jax.experimental.pallas import tpu as pltpu
from jax.experimental.pallas import tpu_sc as plsc

NUM_POSITIONS = 32768
EMB_DIM = 64

IDX_CHUNK = 128


@functools.lru_cache(maxsize=None)
def _positions(seq_len: int) -> np.ndarray:
    with jax.ensure_compile_time_eval():
        pkey = jax.random.key(42)
        perm = np.asarray(jax.random.permutation(pkey, NUM_POSITIONS))
    return np.sort(perm[:seq_len]).astype(np.int32)


@functools.lru_cache(maxsize=None)
def _build_sc_gather(L: int, D: int):
    info = plsc.get_sparse_core_info()
    num_workers = info.num_cores * info.num_subcores
    assert L % (num_workers * IDX_CHUNK) == 0
    rows_per_worker = L // num_workers
    chunks = rows_per_worker // IDX_CHUNK
    mesh = plsc.VectorSubcoreMesh(core_axis_name="c", subcore_axis_name="s")

    @functools.partial(
        pl.kernel,
        out_type=jax.ShapeDtypeStruct((L, D), jnp.float32),
        mesh=mesh,
        scratch_types=[
            pltpu.VMEM((chunks, IDX_CHUNK), jnp.int32),
            pltpu.VMEM((rows_per_worker, D), jnp.float32),
            pltpu.SemaphoreType.DMA,
        ],
        compiler_params=pltpu.CompilerParams(use_tc_tiling_on_sc=False),
    )
    def sc_gather(idx_hbm, table_hbm, out_hbm, idx_v, rows_v, sem):
        wid = lax.axis_index("s") * info.num_cores + lax.axis_index("c")
        base = wid * rows_per_worker
        pltpu.sync_copy(idx_hbm.at[pl.ds(wid * chunks, chunks)], idx_v)
        gathers = [
            pltpu.async_copy(
                table_hbm.at[idx_v.at[j]],
                rows_v.at[pl.ds(j * IDX_CHUNK, IDX_CHUNK)],
                sem,
            )
            for j in range(chunks)
        ]
        for g in gathers:
            g.wait()
        pltpu.async_copy(rows_v, out_hbm.at[pl.ds(base, rows_per_worker)],
                         sem).wait()

    def run(table):
        idx = jnp.asarray(_positions(L).reshape(-1, IDX_CHUNK))
        return sc_gather(idx, table)

    return run


def kernel(x, table):
    B, L = x.shape
    D = table.shape[1]
    rows = _build_sc_gather(L, D)(table)
    return jnp.broadcast_to(rows[None], (B, L, D))
